# baseline (speedup 1.0000x reference)
import functools

import jax
import jax.numpy as jnp
from jax import lax
from jax.experimental import pallas as pl
from jax.experimental.pallas import tpu as pltpu
from jax.experimental.pallas import tpu_sc as plsc

B, N, D = 4096, 19, 512
R = 4
STARTS = (0, 7, 12, 17)
ENDS = (7, 12, 17, 19)
SCALES = (1.0 / 7.0, 1.0 / 5.0, 1.0 / 5.0, 1.0 / 2.0)

NC, NS = 2, 16
NW = NC * NS
BPW = B // NW
CBB = 4
NUNIT = BPW // CBB
LANES = 16
DCHUNKS = D // LANES


def _compute_unit(ibuf, obuf):
    def bbody(b, _):
        for dc in range(DCHUNKS):
            off = dc * LANES
            v = [ibuf[c, b, pl.ds(off, LANES)] for c in range(N)]
            for r in range(R):
                acc = v[STARTS[r]]
                for c in range(STARTS[r] + 1, ENDS[r]):
                    acc = acc + v[c]
                obuf[b, r, pl.ds(off, LANES)] = acc * jnp.float32(SCALES[r])
        return _
    lax.fori_loop(0, CBB, bbody, None)


def _pool_body(xt_hbm, out_hbm, in0, in1, ob0, ob1, isem0, isem1, osem0, osem1):
    wid = lax.axis_index("s") * NC + lax.axis_index("c")
    base = wid * BPW

    def in_slice(u):
        b0 = base + u * CBB
        return xt_hbm.at[:, pl.ds(b0, CBB), :]

    def out_slice(u):
        b0 = base + u * CBB
        return out_hbm.at[pl.ds(b0, CBB), :, :]

    def start_in(u, buf, sem):
        pltpu.async_copy(in_slice(u), buf, sem)

    def wait_in(u, buf, sem):
        pltpu.make_async_copy(in_slice(u), buf, sem).wait()

    def start_out(u, buf, sem):
        pltpu.async_copy(buf, out_slice(u), sem)

    def wait_out(u, buf, sem):
        pltpu.make_async_copy(buf, out_slice(u), sem).wait()

    start_in(0, in0, isem0)

    def ubody(h, _):
        u = h * 2
        wait_in(u, in0, isem0)

        @pl.when(u + 1 < NUNIT)
        def _():
            start_in(u + 1, in1, isem1)

        @pl.when(u >= 2)
        def _():
            wait_out(u - 2, ob0, osem0)

        _compute_unit(in0, ob0)
        start_out(u, ob0, osem0)

        wait_in(u + 1, in1, isem1)

        @pl.when(u + 2 < NUNIT)
        def _():
            start_in(u + 2, in0, isem0)

        @pl.when(u >= 2)
        def _():
            wait_out(u - 1, ob1, osem1)

        _compute_unit(in1, ob1)
        start_out(u + 1, ob1, osem1)
        return _

    lax.fori_loop(0, NUNIT // 2, ubody, None)
    wait_out(NUNIT - 2, ob0, osem0)
    wait_out(NUNIT - 1, ob1, osem1)


_pool = functools.partial(
    pl.kernel,
    out_type=jax.ShapeDtypeStruct((B, R, D), jnp.float32),
    mesh=plsc.VectorSubcoreMesh(core_axis_name="c", subcore_axis_name="s"),
    scratch_types=[
        pltpu.VMEM((N, CBB, D), jnp.float32),
        pltpu.VMEM((N, CBB, D), jnp.float32),
        pltpu.VMEM((CBB, R, D), jnp.float32),
        pltpu.VMEM((CBB, R, D), jnp.float32),
        pltpu.SemaphoreType.DMA,
        pltpu.SemaphoreType.DMA,
        pltpu.SemaphoreType.DMA,
        pltpu.SemaphoreType.DMA,
    ],
)(_pool_body)


@jax.jit
def kernel(node_embeddings):
    x_t = jnp.transpose(node_embeddings, (1, 0, 2))
    return _pool(x_t)


# speedup vs baseline: 1.3147x; 1.3147x over previous
import functools

import jax
import jax.numpy as jnp
from jax import lax
from jax.experimental import pallas as pl
from jax.experimental.pallas import tpu as pltpu
from jax.experimental.pallas import tpu_sc as plsc

B, N, D = 4096, 19, 512
R = 4
STARTS = (0, 7, 12, 17)
ENDS = (7, 12, 17, 19)
SCALES = (1.0 / 7.0, 1.0 / 5.0, 1.0 / 5.0, 1.0 / 2.0)

B_SC = 1280
B_TC = B - B_SC

NC, NS = 2, 16
NW = NC * NS
BPW = B_SC // NW
CBB = 8
DHALF = 256
NUNIT = (BPW // CBB) * (D // DHALF)
LANES = 16
DCHUNKS = DHALF // LANES


def _compute_unit(ibuf, obuf):
    def bbody(b, _):
        for dc in range(DCHUNKS):
            off = dc * LANES
            v = [ibuf[c, b, pl.ds(off, LANES)] for c in range(N)]
            for r in range(R):
                acc = v[STARTS[r]]
                for c in range(STARTS[r] + 1, ENDS[r]):
                    acc = acc + v[c]
                obuf[b, r, pl.ds(off, LANES)] = acc * jnp.float32(SCALES[r])
        return _
    lax.fori_loop(0, CBB, bbody, None)


def _pool_body(xt_hbm, out_hbm, in0, in1, ob0, ob1, isem0, isem1, osem0, osem1):
    wid = lax.axis_index("s") * NC + lax.axis_index("c")
    base = wid * BPW

    def in_slice(u):
        b0 = base + (u // 2) * CBB
        d0 = (u % 2) * DHALF
        return xt_hbm.at[:, pl.ds(b0, CBB), pl.ds(d0, DHALF)]

    def out_slice(u):
        b0 = base + (u // 2) * CBB
        d0 = (u % 2) * DHALF
        return out_hbm.at[pl.ds(b0, CBB), :, pl.ds(d0, DHALF)]

    def start_in(u, buf, sem):
        pltpu.async_copy(in_slice(u), buf, sem)

    def wait_in(u, buf, sem):
        pltpu.make_async_copy(in_slice(u), buf, sem).wait()

    def start_out(u, buf, sem):
        pltpu.async_copy(buf, out_slice(u), sem)

    def wait_out(u, buf, sem):
        pltpu.make_async_copy(buf, out_slice(u), sem).wait()

    start_in(0, in0, isem0)

    def ubody(h, _):
        u = h * 2
        wait_in(u, in0, isem0)

        @pl.when(u + 1 < NUNIT)
        def _():
            start_in(u + 1, in1, isem1)

        @pl.when(u >= 2)
        def _():
            wait_out(u - 2, ob0, osem0)

        _compute_unit(in0, ob0)
        start_out(u, ob0, osem0)

        wait_in(u + 1, in1, isem1)

        @pl.when(u + 2 < NUNIT)
        def _():
            start_in(u + 2, in0, isem0)

        @pl.when(u >= 2)
        def _():
            wait_out(u - 1, ob1, osem1)

        _compute_unit(in1, ob1)
        start_out(u + 1, ob1, osem1)
        return _

    lax.fori_loop(0, NUNIT // 2, ubody, None)
    wait_out(NUNIT - 2, ob0, osem0)
    wait_out(NUNIT - 1, ob1, osem1)


_pool_sc = functools.partial(
    pl.kernel,
    out_type=jax.ShapeDtypeStruct((B_SC, R, D), jnp.float32),
    mesh=plsc.VectorSubcoreMesh(core_axis_name="c", subcore_axis_name="s"),
    scratch_types=[
        pltpu.VMEM((N, CBB, DHALF), jnp.float32),
        pltpu.VMEM((N, CBB, DHALF), jnp.float32),
        pltpu.VMEM((CBB, R, DHALF), jnp.float32),
        pltpu.VMEM((CBB, R, DHALF), jnp.float32),
        pltpu.SemaphoreType.DMA,
        pltpu.SemaphoreType.DMA,
        pltpu.SemaphoreType.DMA,
        pltpu.SemaphoreType.DMA,
    ],
)(_pool_body)


BT = 256
TC_OFF = B_SC // BT


def _tc_body(x_ref, o_ref):
    for r in range(R):
        acc = x_ref[STARTS[r]]
        for c in range(STARTS[r] + 1, ENDS[r]):
            acc = acc + x_ref[c]
        o_ref[:, r, :] = acc * jnp.float32(SCALES[r])


_pool_tc = pl.pallas_call(
    _tc_body,
    grid=(B_TC // BT,),
    in_specs=[pl.BlockSpec((N, BT, D), lambda i: (0, i + TC_OFF, 0))],
    out_specs=pl.BlockSpec((BT, R, D), lambda i: (i + TC_OFF, 0, 0)),
    out_shape=jax.ShapeDtypeStruct((B, R, D), jnp.float32),
)


@jax.jit
def kernel(node_embeddings):
    x_t = jnp.transpose(node_embeddings, (1, 0, 2))
    sc_out = _pool_sc(x_t)
    tc_out = _pool_tc(x_t)
    return lax.dynamic_update_slice(tc_out, sc_out, (0, 0, 0))
